# Initial kernel scaffold; baseline (speedup 1.0000x reference)
import functools

import jax
import jax.numpy as jnp
from jax import lax
from jax.experimental import pallas as pl
from jax.experimental.pallas import tpu as pltpu
from jax.experimental.pallas import tpu_sc as plsc

VOCAB = 1000000
EMBED = 64
BATCH = 16384
HIST = 200

NC = 2
NS = 16
L = 16
NW = NC * NS

ROWS_PER_W = BATCH // NW
BR = 64
NBLK = ROWS_PER_W // BR
NG = EMBED // L
C0 = 128
C1 = HIST - C0


def _body(idx_hbm, table_hbm, out_hbm, idx_v, rows_v, out_v, sem):
    wid = lax.axis_index("s") * NC + lax.axis_index("c")
    base = wid * ROWS_PER_W

    def blk_body(blk, carry):
        row0 = base + blk * BR
        pltpu.sync_copy(idx_hbm.at[pl.ds(row0, BR)], idx_v)

        def row_body(r, carry):
            cp0 = pltpu.async_copy(
                table_hbm.at[idx_v.at[r, pl.ds(0, C0)]],
                rows_v.at[pl.ds(0, C0)], sem)
            cp1 = pltpu.async_copy(
                table_hbm.at[idx_v.at[r, pl.ds(C0, C1)]],
                rows_v.at[pl.ds(C0, C1)], sem)
            cp0.wait()
            cp1.wait()

            init = tuple(rows_v[0, pl.ds(g * L, L)] for g in range(NG))

            def red(i, accs):
                return tuple(
                    jnp.maximum(a, rows_v[i, pl.ds(g * L, L)])
                    for g, a in enumerate(accs))

            accs = lax.fori_loop(1, HIST, red, init)
            for g in range(NG):
                x = accs[g]
                e = jnp.exp(x + x)
                out_v[r, pl.ds(g * L, L)] = 1.0 - 2.0 / (e + 1.0)
            return carry

        lax.fori_loop(0, BR, row_body, 0)
        pltpu.sync_copy(out_v, out_hbm.at[pl.ds(row0, BR)])
        return carry

    lax.fori_loop(0, NBLK, blk_body, 0)


@jax.jit
def kernel(input, table):
    idx = input.astype(jnp.int32)
    mesh = plsc.VectorSubcoreMesh(
        core_axis_name="c", subcore_axis_name="s",
        num_cores=NC, num_subcores=NS)
    k = pl.kernel(
        _body,
        out_type=jax.ShapeDtypeStruct((BATCH, EMBED), jnp.float32),
        mesh=mesh,
        scratch_types=[
            pltpu.VMEM((BR, HIST), jnp.int32),
            pltpu.VMEM((HIST, EMBED), jnp.float32),
            pltpu.VMEM((BR, EMBED), jnp.float32),
            pltpu.SemaphoreType.DMA,
        ],
    )
    return k(idx, table)


# speedup vs baseline: 1.9989x; 1.9989x over previous
import functools

import jax
import jax.numpy as jnp
from jax import lax
from jax.experimental import pallas as pl
from jax.experimental.pallas import tpu as pltpu
from jax.experimental.pallas import tpu_sc as plsc

VOCAB = 1000000
EMBED = 64
BATCH = 16384
HIST = 200

NC = 2
NS = 16
L = 16
NW = NC * NS

ROWS_PER_W = BATCH // NW
BR = 64
NBLK = ROWS_PER_W // BR
NG = EMBED // L
C0 = 128
C1 = HIST - C0


def _body(idx_hbm, table_hbm, out_hbm, idx_v, rows_v, out_v, sem):
    wid = lax.axis_index("s") * NC + lax.axis_index("c")
    base = wid * ROWS_PER_W

    def blk_body(blk, carry):
        row0 = base + blk * BR
        pltpu.sync_copy(idx_hbm.at[pl.ds(row0, BR)], idx_v)

        def row_body(r, carry):
            cp0 = pltpu.async_copy(
                table_hbm.at[idx_v.at[r, pl.ds(0, C0)]],
                rows_v.at[pl.ds(0, C0)], sem)
            cp1 = pltpu.async_copy(
                table_hbm.at[idx_v.at[r, pl.ds(C0, C1)]],
                rows_v.at[pl.ds(C0, C1)], sem)
            cp0.wait()
            cp1.wait()

            init = tuple(rows_v[0, pl.ds(g * L, L)] for g in range(NG))

            def red(i, accs):
                return tuple(
                    jnp.maximum(a, rows_v[i, pl.ds(g * L, L)])
                    for g, a in enumerate(accs))

            accs = lax.fori_loop(1, HIST, red, init)
            for g in range(NG):
                x = accs[g]
                e = jnp.exp(x + x)
                out_v[r, pl.ds(g * L, L)] = 1.0 - 2.0 / (e + 1.0)
            return carry

        lax.fori_loop(0, BR, row_body, 0)
        pltpu.sync_copy(out_v, out_hbm.at[pl.ds(row0, BR)])
        return carry

    lax.fori_loop(0, NBLK, blk_body, 0)


@jax.jit
def kernel(input, table):
    idx = input.astype(jnp.int32)
    mesh = plsc.VectorSubcoreMesh(
        core_axis_name="c", subcore_axis_name="s",
        num_cores=NC, num_subcores=NS)
    k = pl.kernel(
        _body,
        out_type=jax.ShapeDtypeStruct((BATCH, EMBED), jnp.float32),
        mesh=mesh,
        scratch_types=[
            pltpu.VMEM((BR, HIST), jnp.int32),
            pltpu.VMEM((HIST, EMBED), jnp.float32),
            pltpu.VMEM((BR, EMBED), jnp.float32),
            pltpu.SemaphoreType.DMA,
        ],
        compiler_params=pltpu.CompilerParams(use_tc_tiling_on_sc=False),
    )
    return k(idx, table)


# speedup vs baseline: 2.9244x; 1.4630x over previous
import functools

import jax
import jax.numpy as jnp
from jax import lax
from jax.experimental import pallas as pl
from jax.experimental.pallas import tpu as pltpu
from jax.experimental.pallas import tpu_sc as plsc

VOCAB = 1000000
EMBED = 64
BATCH = 16384
HIST = 200

NC = 2
NS = 16
L = 16
NW = NC * NS

ROWS_PER_W = BATCH // NW
BR = 64
NBLK = ROWS_PER_W // BR
NG = EMBED // L
C0 = 128
C1 = HIST - C0


UNROLL = 4


def _body(idx_hbm, table_hbm, out_hbm, idx_v, rows0_v, rows1_v, out_v,
          sem0, sem1):
    wid = lax.axis_index("s") * NC + lax.axis_index("c")
    base = wid * ROWS_PER_W
    bufs = (rows0_v, rows1_v)
    sems = (sem0, sem1)

    def start_gather(r, buf, sem):
        pltpu.async_copy(
            table_hbm.at[idx_v.at[r, pl.ds(0, C0)]],
            buf.at[pl.ds(0, C0)], sem)
        pltpu.async_copy(
            table_hbm.at[idx_v.at[r, pl.ds(C0, C1)]],
            buf.at[pl.ds(C0, C1)], sem)

    def wait_gather(r, buf, sem):
        pltpu.make_async_copy(
            table_hbm.at[idx_v.at[r, pl.ds(0, C0)]],
            buf.at[pl.ds(0, C0)], sem).wait()
        pltpu.make_async_copy(
            table_hbm.at[idx_v.at[r, pl.ds(C0, C1)]],
            buf.at[pl.ds(C0, C1)], sem).wait()

    def reduce_row(r, buf):
        init = tuple(
            jnp.full((L,), -jnp.inf, jnp.float32) for _ in range(NG))

        def red(i, accs):
            accs = list(accs)
            for u in range(UNROLL):
                row = i * UNROLL + u
                for g in range(NG):
                    accs[g] = jnp.maximum(accs[g], buf[row, pl.ds(g * L, L)])
            return tuple(accs)

        accs = lax.fori_loop(0, HIST // UNROLL, red, init)
        for g in range(NG):
            x = accs[g]
            e = jnp.exp(x + x)
            out_v[r, pl.ds(g * L, L)] = 1.0 - 2.0 / (e + 1.0)

    def blk_body(blk, carry):
        row0 = base + blk * BR
        pltpu.sync_copy(idx_hbm.at[pl.ds(row0, BR)], idx_v)
        start_gather(0, bufs[0], sems[0])

        def pair_body(j, carry):
            r0 = 2 * j
            start_gather(r0 + 1, bufs[1], sems[1])
            wait_gather(r0, bufs[0], sems[0])
            reduce_row(r0, bufs[0])

            @pl.when(j < BR // 2 - 1)
            def _():
                start_gather(r0 + 2, bufs[0], sems[0])

            wait_gather(r0 + 1, bufs[1], sems[1])
            reduce_row(r0 + 1, bufs[1])
            return carry

        lax.fori_loop(0, BR // 2, pair_body, 0)
        pltpu.sync_copy(out_v, out_hbm.at[pl.ds(row0, BR)])
        return carry

    lax.fori_loop(0, NBLK, blk_body, 0)


@jax.jit
def kernel(input, table):
    idx = input.astype(jnp.int32)
    mesh = plsc.VectorSubcoreMesh(
        core_axis_name="c", subcore_axis_name="s",
        num_cores=NC, num_subcores=NS)
    k = pl.kernel(
        _body,
        out_type=jax.ShapeDtypeStruct((BATCH, EMBED), jnp.float32),
        mesh=mesh,
        scratch_types=[
            pltpu.VMEM((BR, HIST), jnp.int32),
            pltpu.VMEM((HIST, EMBED), jnp.float32),
            pltpu.VMEM((HIST, EMBED), jnp.float32),
            pltpu.VMEM((BR, EMBED), jnp.float32),
            pltpu.SemaphoreType.DMA,
            pltpu.SemaphoreType.DMA,
        ],
        compiler_params=pltpu.CompilerParams(use_tc_tiling_on_sc=False),
    )
    return k(idx, table)
